# baseline (speedup 1.0000x reference)
import functools

import jax
import jax.numpy as jnp
from jax import lax
from jax.experimental import pallas as pl
from jax.experimental.pallas import tpu as pltpu
from jax.experimental.pallas import tpu_sc as plsc

N = 10000
D = 128
E = 320000
W = 6
H = 128
EPS = 1e-3

NC = 2
NS = 16
HD = D // NC
NP = 10240
RPT = NP // NS
EPT = E // NS
CH = 125
CG = 8
NSUP = EPT // (CH * CG)
NBUF = 6
NIF = 5
DW = 16

BSB = 512
BSC = 512

F32 = jnp.float32


@functools.lru_cache(maxsize=None)
def _make_mpnn(with_deg):
  mesh = plsc.VectorSubcoreMesh(core_axis_name="c", subcore_axis_name="s")
  out_type = [jax.ShapeDtypeStruct((W, NP, D), F32)]
  scratch = [
      pltpu.VMEM((CG, CH), jnp.int32),
      pltpu.VMEM((CG, CH), jnp.int32),
      pltpu.VMEM((NBUF, CH, HD), F32),
      pltpu.VMEM_SHARED((NP, HD), F32),
      pltpu.SemaphoreType.DMA,
      pltpu.SemaphoreType.DMA,
      pltpu.SemaphoreType.DMA,
      pltpu.SemaphoreType.DMA,
      pltpu.SemaphoreType.DMA,
  ]
  if with_deg:
    out_type.append(jax.ShapeDtypeStruct((NC, W, NP, DW), F32))
    scratch += [pltpu.VMEM((CH, DW), F32), pltpu.VMEM_SHARED((NP, DW), F32)]

  def body(table, src_h, dst_h, zrow_h, *rest):
    if with_deg:
      (zdeg_h, half_h, out_p, out_d, src_v, dst_v, rows_v, acc,
       sem0, sem1, sem2, sem3, sem4, ones_v, dacc) = rest
    else:
      (out_p, src_v, dst_v, rows_v, acc,
       sem0, sem1, sem2, sem3, sem4) = rest
    sems = (sem0, sem1, sem2, sem3, sem4)
    c = lax.axis_index("c")
    s = lax.axis_index("s")
    row0 = s * RPT
    if with_deg:
      pltpu.sync_copy(half_h, ones_v)

    def win_body(w, carry):
      pltpu.sync_copy(zrow_h, acc.at[pl.ds(row0, RPT)])
      if with_deg:
        pltpu.sync_copy(zdeg_h, dacc.at[pl.ds(row0, RPT)])
      plsc.subcore_barrier()

      def chunk_body(j, carry2):
        pltpu.sync_copy(src_h.at[c, w, s, j], src_v)
        pltpu.sync_copy(dst_h.at[w, s, j], dst_v)
        descs = [None] * CG
        for jj in range(NIF):
          descs[jj] = pltpu.async_copy(
              table.at[src_v.at[jj]], rows_v.at[jj % NBUF], sems[jj % NIF])
        for jj in range(CG):
          descs[jj].wait()
          if jj + NIF < CG:
            descs[jj + NIF] = pltpu.async_copy(
                table.at[src_v.at[jj + NIF]], rows_v.at[(jj + NIF) % NBUF],
                sems[jj % NIF])
          pltpu.sync_copy(rows_v.at[jj % NBUF], acc.at[dst_v.at[jj]],
                          add=True)
          if with_deg:
            pltpu.sync_copy(ones_v, dacc.at[dst_v.at[jj]], add=True)
        return carry2

      lax.fori_loop(0, NSUP, chunk_body, 0)
      plsc.subcore_barrier()
      pltpu.sync_copy(acc.at[pl.ds(row0, RPT)],
                      out_p.at[w, pl.ds(row0, RPT), pl.ds(c * HD, HD)])
      if with_deg:
        pltpu.sync_copy(dacc.at[pl.ds(row0, RPT)],
                        out_d.at[c, w, pl.ds(row0, RPT)])
      return carry

    lax.fori_loop(0, W, win_body, 0)

  return pl.kernel(
      body,
      out_type=tuple(out_type) if with_deg else out_type[0],
      mesh=mesh,
      compiler_params=pltpu.CompilerParams(use_tc_tiling_on_sc=False),
      scratch_types=scratch,
  )


def _bn1_body(p_ref, d_ref, g_ref, b_ref, m_ref, v_ref, o_ref):
  p = p_ref[0]
  deg = d_ref[0, 0] + d_ref[1, 0]
  degc = jnp.maximum(deg[:, :1], 1.0)
  h = jnp.maximum(p / degc, 0.0)
  o_ref[0] = ((h - m_ref[0, 0]) * lax.rsqrt(v_ref[0, 0] + EPS) * g_ref[0, 0]
              + b_ref[0, 0])


def _bn1(P, Dg, g, b, m, v):
  g, b, m, v = (x[:, None, :] for x in (g, b, m, v))
  pspec = pl.BlockSpec((1, BSB, D), lambda w, i: (w, i, 0))
  dspec = pl.BlockSpec((NC, 1, BSB, DW), lambda w, i: (0, w, i, 0))
  wspec = pl.BlockSpec((1, 1, D), lambda w, i: (w, 0, 0))
  return pl.pallas_call(
      _bn1_body,
      grid=(W, NP // BSB),
      in_specs=[pspec, dspec, wspec, wspec, wspec, wspec],
      out_specs=pl.BlockSpec((1, BSB, D), lambda w, i: (w, i, 0)),
      out_shape=jax.ShapeDtypeStruct((W, NP, D), F32),
  )(P, Dg, g, b, m, v)


def _sigmoid(x):
  return 1.0 / (1.0 + jnp.exp(-x))


def _head_body(h1_ref, p2_ref, g2_ref, b2_ref, m2_ref, v2_ref,
               w1_ref, u1_ref, b1_ref, w2_ref, u2_ref, bb2_ref,
               wd_ref, bd_ref, o_ref):
  xs = []
  for w in range(W):
    h2 = jnp.maximum(p2_ref[w], 0.0)
    h2 = ((h2 - m2_ref[w]) * lax.rsqrt(v2_ref[w] + EPS) * g2_ref[w]
          + b2_ref[w])
    xs.append(jnp.concatenate([h1_ref[w], h2], axis=1))

  w1 = w1_ref[...]
  u1 = u1_ref[...]
  b1 = b1_ref[0]
  h = jnp.zeros((BSC, H), F32)
  c = jnp.zeros((BSC, H), F32)
  hs = []
  for t in range(W):
    z = (jnp.dot(xs[t], w1, preferred_element_type=F32)
         + jnp.dot(h, u1, preferred_element_type=F32) + b1)
    c = _sigmoid(z[:, H:2 * H]) * c + _sigmoid(z[:, :H]) * jnp.tanh(
        z[:, 2 * H:3 * H])
    h = _sigmoid(z[:, 3 * H:]) * jnp.tanh(c)
    hs.append(h)

  w2 = w2_ref[...]
  u2 = u2_ref[...]
  b2 = bb2_ref[0]
  h = jnp.zeros((BSC, H), F32)
  c = jnp.zeros((BSC, H), F32)
  for t in range(W):
    z = (jnp.dot(hs[t], w2, preferred_element_type=F32)
         + jnp.dot(h, u2, preferred_element_type=F32) + b2)
    c = _sigmoid(z[:, H:2 * H]) * c + _sigmoid(z[:, :H]) * jnp.tanh(
        z[:, 2 * H:3 * H])
    h = _sigmoid(z[:, 3 * H:]) * jnp.tanh(c)

  o_ref[...] = jnp.maximum(
      jnp.dot(h, wd_ref[...], preferred_element_type=F32) + bd_ref[0], 0.0)


def _head(h1, P2, g2, b2, m2, v2, W1, U1, b1, W2, U2, bb2, Wdp, bdp):
  full = lambda *shape: pl.BlockSpec(shape, lambda i: (0,) * len(shape))
  return pl.pallas_call(
      _head_body,
      grid=(NP // BSC,),
      in_specs=[
          pl.BlockSpec((W, BSC, D), lambda i: (0, i, 0)),
          pl.BlockSpec((W, BSC, D), lambda i: (0, i, 0)),
          full(W, D), full(W, D), full(W, D), full(W, D),
          full(2 * D, 4 * H), full(H, 4 * H), full(1, 4 * H),
          full(H, 4 * H), full(H, 4 * H), full(1, 4 * H),
          full(H, 128), full(1, 128),
      ],
      out_specs=pl.BlockSpec((BSC, 128), lambda i: (i, 0)),
      out_shape=jax.ShapeDtypeStruct((NP, 128), F32),
  )(h1, P2, g2, b2, m2, v2, W1, U1, b1, W2, U2, bb2, Wdp, bdp)


def _split_idx(base):
  two = base * 2
  stacked = jnp.stack([two, two + 1])
  return stacked.reshape(NC, W, NS, NSUP, CG, CH)


def kernel(X, edge_index, bn1_gamma, bn1_beta, bn1_mean, bn1_var,
           bn2_gamma, bn2_beta, bn2_mean, bn2_var,
           W1, U1, b1, W2, U2, b2, Wd, bd):
  src = edge_index[:, 0, :]
  dst = edge_index[:, 1, :]
  woff = jnp.arange(W, dtype=jnp.int32)[:, None]
  src1 = _split_idx(src + woff * N)
  src2 = _split_idx(src + woff * NP)
  dstr = dst.reshape(W, NS, NSUP, CG, CH)

  zrow = jnp.zeros((RPT, HD), F32)
  zdeg = jnp.zeros((RPT, DW), F32)
  half = jnp.full((CH, DW), 0.5, F32)

  P1, Dg = _make_mpnn(True)(X.reshape(W * N * NC, HD), src1, dstr,
                            zrow, zdeg, half)
  h1 = _bn1(P1, Dg, bn1_gamma, bn1_beta, bn1_mean, bn1_var)
  P2 = _make_mpnn(False)(h1.reshape(W * NP * NC, HD), src2, dstr, zrow)

  Wdp = jnp.pad(Wd, ((0, 0), (0, 127)))
  bdp = jnp.pad(bd, (0, 127))[None, :]
  out = _head(h1, P2, bn2_gamma, bn2_beta, bn2_mean, bn2_var,
              W1, U1, b1[None, :], W2, U2, b2[None, :], Wdp, bdp)
  return out[:N, :1]


# speedup vs baseline: 1.2072x; 1.2072x over previous
import functools

import jax
import jax.numpy as jnp
from jax import lax
from jax.experimental import pallas as pl
from jax.experimental.pallas import tpu as pltpu
from jax.experimental.pallas import tpu_sc as plsc

N = 10000
D = 128
E = 320000
W = 6
H = 128
EPS = 1e-3

NC = 2
NS = 16
HD = D // NC
NP = 10240
RPT = NP // NS
EPT = E // NS
CH = 125
CG = 16
NSUP = EPT // (CH * CG)
NBUF = 4
NIF = 3
DW = 16

BSB = 512
BSC = 512

F32 = jnp.float32


@functools.lru_cache(maxsize=None)
def _make_mpnn(with_deg):
  mesh = plsc.VectorSubcoreMesh(core_axis_name="c", subcore_axis_name="s")
  out_type = [jax.ShapeDtypeStruct((W, NP, D), F32)]
  scratch = [
      pltpu.VMEM((CG, CH), jnp.int32),
      pltpu.VMEM((CG, CH), jnp.int32),
      pltpu.VMEM((NBUF, CH, HD), F32),
      pltpu.VMEM_SHARED((NP, HD), F32),
      pltpu.SemaphoreType.DMA,
      pltpu.SemaphoreType.DMA,
      pltpu.SemaphoreType.DMA,
  ]
  if with_deg:
    out_type.append(jax.ShapeDtypeStruct((NC, W, NP, DW), F32))
    scratch += [pltpu.VMEM((CH, DW), F32), pltpu.VMEM_SHARED((NP, DW), F32)]

  def body(table, src_h, dst_h, zrow_h, *rest):
    if with_deg:
      (zdeg_h, half_h, out_p, out_d, src_v, dst_v, rows_v, acc,
       sem0, sem1, sem2, ones_v, dacc) = rest
    else:
      (out_p, src_v, dst_v, rows_v, acc, sem0, sem1, sem2) = rest
    sems = (sem0, sem1, sem2)
    c = lax.axis_index("c")
    s = lax.axis_index("s")
    row0 = s * RPT
    if with_deg:
      pltpu.sync_copy(half_h, ones_v)

    def win_body(w, carry):
      pltpu.sync_copy(zrow_h, acc.at[pl.ds(row0, RPT)])
      if with_deg:
        pltpu.sync_copy(zdeg_h, dacc.at[pl.ds(row0, RPT)])
      plsc.subcore_barrier()

      def chunk_body(j, carry2):
        pltpu.sync_copy(src_h.at[c, w, s, j], src_v)
        pltpu.sync_copy(dst_h.at[w, s, j], dst_v)
        descs = [None] * CG
        for jj in range(NIF):
          descs[jj] = pltpu.async_copy(
              table.at[src_v.at[jj]], rows_v.at[jj % NBUF], sems[jj % NIF])
        for jj in range(CG):
          descs[jj].wait()
          if jj + NIF < CG:
            descs[jj + NIF] = pltpu.async_copy(
                table.at[src_v.at[jj + NIF]], rows_v.at[(jj + NIF) % NBUF],
                sems[jj % NIF])
          pltpu.sync_copy(rows_v.at[jj % NBUF], acc.at[dst_v.at[jj]],
                          add=True)
          if with_deg:
            pltpu.sync_copy(ones_v, dacc.at[dst_v.at[jj]], add=True)
        return carry2

      lax.fori_loop(0, NSUP, chunk_body, 0)
      plsc.subcore_barrier()
      pltpu.sync_copy(acc.at[pl.ds(row0, RPT)],
                      out_p.at[w, pl.ds(row0, RPT), pl.ds(c * HD, HD)])
      if with_deg:
        pltpu.sync_copy(dacc.at[pl.ds(row0, RPT)],
                        out_d.at[c, w, pl.ds(row0, RPT)])
      return carry

    lax.fori_loop(0, W, win_body, 0)

  return pl.kernel(
      body,
      out_type=tuple(out_type) if with_deg else out_type[0],
      mesh=mesh,
      compiler_params=pltpu.CompilerParams(use_tc_tiling_on_sc=False),
      scratch_types=scratch,
  )


def _bn1_body(p_ref, d_ref, g_ref, b_ref, m_ref, v_ref, o_ref):
  p = p_ref[0]
  deg = d_ref[0, 0] + d_ref[1, 0]
  degc = jnp.maximum(deg[:, :1], 1.0)
  h = jnp.maximum(p / degc, 0.0)
  o_ref[0] = ((h - m_ref[0, 0]) * lax.rsqrt(v_ref[0, 0] + EPS) * g_ref[0, 0]
              + b_ref[0, 0])


def _bn1(P, Dg, g, b, m, v):
  g, b, m, v = (x[:, None, :] for x in (g, b, m, v))
  pspec = pl.BlockSpec((1, BSB, D), lambda w, i: (w, i, 0))
  dspec = pl.BlockSpec((NC, 1, BSB, DW), lambda w, i: (0, w, i, 0))
  wspec = pl.BlockSpec((1, 1, D), lambda w, i: (w, 0, 0))
  return pl.pallas_call(
      _bn1_body,
      grid=(W, NP // BSB),
      in_specs=[pspec, dspec, wspec, wspec, wspec, wspec],
      out_specs=pl.BlockSpec((1, BSB, D), lambda w, i: (w, i, 0)),
      out_shape=jax.ShapeDtypeStruct((W, NP, D), F32),
  )(P, Dg, g, b, m, v)


def _sigmoid(x):
  return 1.0 / (1.0 + jnp.exp(-x))


def _head_body(h1_ref, p2_ref, g2_ref, b2_ref, m2_ref, v2_ref,
               w1_ref, u1_ref, b1_ref, w2_ref, u2_ref, bb2_ref,
               wd_ref, bd_ref, o_ref):
  xs = []
  for w in range(W):
    h2 = jnp.maximum(p2_ref[w], 0.0)
    h2 = ((h2 - m2_ref[w]) * lax.rsqrt(v2_ref[w] + EPS) * g2_ref[w]
          + b2_ref[w])
    xs.append(jnp.concatenate([h1_ref[w], h2], axis=1))

  w1 = w1_ref[...]
  u1 = u1_ref[...]
  b1 = b1_ref[0]
  h = jnp.zeros((BSC, H), F32)
  c = jnp.zeros((BSC, H), F32)
  hs = []
  for t in range(W):
    z = (jnp.dot(xs[t], w1, preferred_element_type=F32)
         + jnp.dot(h, u1, preferred_element_type=F32) + b1)
    c = _sigmoid(z[:, H:2 * H]) * c + _sigmoid(z[:, :H]) * jnp.tanh(
        z[:, 2 * H:3 * H])
    h = _sigmoid(z[:, 3 * H:]) * jnp.tanh(c)
    hs.append(h)

  w2 = w2_ref[...]
  u2 = u2_ref[...]
  b2 = bb2_ref[0]
  h = jnp.zeros((BSC, H), F32)
  c = jnp.zeros((BSC, H), F32)
  for t in range(W):
    z = (jnp.dot(hs[t], w2, preferred_element_type=F32)
         + jnp.dot(h, u2, preferred_element_type=F32) + b2)
    c = _sigmoid(z[:, H:2 * H]) * c + _sigmoid(z[:, :H]) * jnp.tanh(
        z[:, 2 * H:3 * H])
    h = _sigmoid(z[:, 3 * H:]) * jnp.tanh(c)

  o_ref[...] = jnp.maximum(
      jnp.dot(h, wd_ref[...], preferred_element_type=F32) + bd_ref[0], 0.0)


def _head(h1, P2, g2, b2, m2, v2, W1, U1, b1, W2, U2, bb2, Wdp, bdp):
  full = lambda *shape: pl.BlockSpec(shape, lambda i: (0,) * len(shape))
  return pl.pallas_call(
      _head_body,
      grid=(NP // BSC,),
      in_specs=[
          pl.BlockSpec((W, BSC, D), lambda i: (0, i, 0)),
          pl.BlockSpec((W, BSC, D), lambda i: (0, i, 0)),
          full(W, D), full(W, D), full(W, D), full(W, D),
          full(2 * D, 4 * H), full(H, 4 * H), full(1, 4 * H),
          full(H, 4 * H), full(H, 4 * H), full(1, 4 * H),
          full(H, 128), full(1, 128),
      ],
      out_specs=pl.BlockSpec((BSC, 128), lambda i: (i, 0)),
      out_shape=jax.ShapeDtypeStruct((NP, 128), F32),
  )(h1, P2, g2, b2, m2, v2, W1, U1, b1, W2, U2, bb2, Wdp, bdp)


def _split_idx(base):
  two = base * 2
  stacked = jnp.stack([two, two + 1])
  return stacked.reshape(NC, W, NS, NSUP, CG, CH)


def kernel(X, edge_index, bn1_gamma, bn1_beta, bn1_mean, bn1_var,
           bn2_gamma, bn2_beta, bn2_mean, bn2_var,
           W1, U1, b1, W2, U2, b2, Wd, bd):
  src = edge_index[:, 0, :]
  dst = edge_index[:, 1, :]
  woff = jnp.arange(W, dtype=jnp.int32)[:, None]
  src1 = _split_idx(src + woff * N)
  src2 = _split_idx(src + woff * NP)
  dstr = dst.reshape(W, NS, NSUP, CG, CH)

  zrow = jnp.zeros((RPT, HD), F32)
  zdeg = jnp.zeros((RPT, DW), F32)
  half = jnp.full((CH, DW), 0.5, F32)

  P1, Dg = _make_mpnn(True)(X.reshape(W * N * NC, HD), src1, dstr,
                            zrow, zdeg, half)
  h1 = _bn1(P1, Dg, bn1_gamma, bn1_beta, bn1_mean, bn1_var)
  P2 = _make_mpnn(False)(h1.reshape(W * NP * NC, HD), src2, dstr, zrow)

  Wdp = jnp.pad(Wd, ((0, 0), (0, 127)))
  bdp = jnp.pad(bd, (0, 127))[None, :]
  out = _head(h1, P2, bn2_gamma, bn2_beta, bn2_mean, bn2_var,
              W1, U1, b1[None, :], W2, U2, b2[None, :], Wdp, bdp)
  return out[:N, :1]


# speedup vs baseline: 1.3003x; 1.0771x over previous
import functools

import jax
import jax.numpy as jnp
from jax import lax
from jax.experimental import pallas as pl
from jax.experimental.pallas import tpu as pltpu
from jax.experimental.pallas import tpu_sc as plsc

N = 10000
D = 128
E = 320000
W = 6
H = 128
EPS = 1e-3

NC = 2
NS = 16
HD = D // NC
NP = 10240
RPT = NP // NS
EPT = E // NS
CH = 125
CG = 32
NSUP = EPT // (CH * CG)
NBUF = 4
NIF = 3
DW = 16

BSB = 512
BSC = 512

F32 = jnp.float32


@functools.lru_cache(maxsize=None)
def _make_mpnn(with_deg):
  mesh = plsc.VectorSubcoreMesh(core_axis_name="c", subcore_axis_name="s")
  out_type = [jax.ShapeDtypeStruct((W, NP, D), F32)]
  scratch = [
      pltpu.VMEM((CG, CH), jnp.int32),
      pltpu.VMEM((CG, CH), jnp.int32),
      pltpu.VMEM((NBUF, CH, HD), F32),
      pltpu.VMEM_SHARED((NP, HD), F32),
      pltpu.SemaphoreType.DMA,
      pltpu.SemaphoreType.DMA,
      pltpu.SemaphoreType.DMA,
  ]
  if with_deg:
    out_type.append(jax.ShapeDtypeStruct((NC, W, NP, DW), F32))
    scratch += [pltpu.VMEM((CH, DW), F32), pltpu.VMEM_SHARED((NP, DW), F32)]

  def body(table, src_h, dst_h, zrow_h, *rest):
    if with_deg:
      (zdeg_h, half_h, out_p, out_d, src_v, dst_v, rows_v, acc,
       sem0, sem1, sem2, ones_v, dacc) = rest
    else:
      (out_p, src_v, dst_v, rows_v, acc, sem0, sem1, sem2) = rest
    sems = (sem0, sem1, sem2)
    c = lax.axis_index("c")
    s = lax.axis_index("s")
    row0 = s * RPT
    if with_deg:
      pltpu.sync_copy(half_h, ones_v)

    def win_body(w, carry):
      pltpu.sync_copy(zrow_h, acc.at[pl.ds(row0, RPT)])
      if with_deg:
        pltpu.sync_copy(zdeg_h, dacc.at[pl.ds(row0, RPT)])
      plsc.subcore_barrier()

      def chunk_body(j, carry2):
        pltpu.sync_copy(src_h.at[c, w, s, j], src_v)
        pltpu.sync_copy(dst_h.at[w, s, j], dst_v)
        descs = [None] * CG
        for jj in range(NIF):
          descs[jj] = pltpu.async_copy(
              table.at[src_v.at[jj]], rows_v.at[jj % NBUF], sems[jj % NIF])
        for jj in range(CG):
          descs[jj].wait()
          if jj + NIF < CG:
            descs[jj + NIF] = pltpu.async_copy(
                table.at[src_v.at[jj + NIF]], rows_v.at[(jj + NIF) % NBUF],
                sems[jj % NIF])
          pltpu.sync_copy(rows_v.at[jj % NBUF], acc.at[dst_v.at[jj]],
                          add=True)
          if with_deg:
            pltpu.sync_copy(ones_v, dacc.at[dst_v.at[jj]], add=True)
        return carry2

      lax.fori_loop(0, NSUP, chunk_body, 0)
      plsc.subcore_barrier()
      pltpu.sync_copy(acc.at[pl.ds(row0, RPT)],
                      out_p.at[w, pl.ds(row0, RPT), pl.ds(c * HD, HD)])
      if with_deg:
        pltpu.sync_copy(dacc.at[pl.ds(row0, RPT)],
                        out_d.at[c, w, pl.ds(row0, RPT)])
      return carry

    lax.fori_loop(0, W, win_body, 0)

  return pl.kernel(
      body,
      out_type=tuple(out_type) if with_deg else out_type[0],
      mesh=mesh,
      compiler_params=pltpu.CompilerParams(use_tc_tiling_on_sc=False),
      scratch_types=scratch,
  )


def _bn1_body(p_ref, d_ref, g_ref, b_ref, m_ref, v_ref, o_ref):
  p = p_ref[0]
  deg = d_ref[0, 0] + d_ref[1, 0]
  degc = jnp.maximum(deg[:, :1], 1.0)
  h = jnp.maximum(p / degc, 0.0)
  o_ref[0] = ((h - m_ref[0, 0]) * lax.rsqrt(v_ref[0, 0] + EPS) * g_ref[0, 0]
              + b_ref[0, 0])


def _bn1(P, Dg, g, b, m, v):
  g, b, m, v = (x[:, None, :] for x in (g, b, m, v))
  pspec = pl.BlockSpec((1, BSB, D), lambda w, i: (w, i, 0))
  dspec = pl.BlockSpec((NC, 1, BSB, DW), lambda w, i: (0, w, i, 0))
  wspec = pl.BlockSpec((1, 1, D), lambda w, i: (w, 0, 0))
  return pl.pallas_call(
      _bn1_body,
      grid=(W, NP // BSB),
      in_specs=[pspec, dspec, wspec, wspec, wspec, wspec],
      out_specs=pl.BlockSpec((1, BSB, D), lambda w, i: (w, i, 0)),
      out_shape=jax.ShapeDtypeStruct((W, NP, D), F32),
  )(P, Dg, g, b, m, v)


def _sigmoid(x):
  return 1.0 / (1.0 + jnp.exp(-x))


def _head_body(h1_ref, p2_ref, g2_ref, b2_ref, m2_ref, v2_ref,
               w1_ref, u1_ref, b1_ref, w2_ref, u2_ref, bb2_ref,
               wd_ref, bd_ref, o_ref):
  xs = []
  for w in range(W):
    h2 = jnp.maximum(p2_ref[w], 0.0)
    h2 = ((h2 - m2_ref[w]) * lax.rsqrt(v2_ref[w] + EPS) * g2_ref[w]
          + b2_ref[w])
    xs.append(jnp.concatenate([h1_ref[w], h2], axis=1))

  w1 = w1_ref[...]
  u1 = u1_ref[...]
  b1 = b1_ref[0]
  h = jnp.zeros((BSC, H), F32)
  c = jnp.zeros((BSC, H), F32)
  hs = []
  for t in range(W):
    z = (jnp.dot(xs[t], w1, preferred_element_type=F32)
         + jnp.dot(h, u1, preferred_element_type=F32) + b1)
    c = _sigmoid(z[:, H:2 * H]) * c + _sigmoid(z[:, :H]) * jnp.tanh(
        z[:, 2 * H:3 * H])
    h = _sigmoid(z[:, 3 * H:]) * jnp.tanh(c)
    hs.append(h)

  w2 = w2_ref[...]
  u2 = u2_ref[...]
  b2 = bb2_ref[0]
  h = jnp.zeros((BSC, H), F32)
  c = jnp.zeros((BSC, H), F32)
  for t in range(W):
    z = (jnp.dot(hs[t], w2, preferred_element_type=F32)
         + jnp.dot(h, u2, preferred_element_type=F32) + b2)
    c = _sigmoid(z[:, H:2 * H]) * c + _sigmoid(z[:, :H]) * jnp.tanh(
        z[:, 2 * H:3 * H])
    h = _sigmoid(z[:, 3 * H:]) * jnp.tanh(c)

  o_ref[...] = jnp.maximum(
      jnp.dot(h, wd_ref[...], preferred_element_type=F32) + bd_ref[0], 0.0)


def _head(h1, P2, g2, b2, m2, v2, W1, U1, b1, W2, U2, bb2, Wdp, bdp):
  full = lambda *shape: pl.BlockSpec(shape, lambda i: (0,) * len(shape))
  return pl.pallas_call(
      _head_body,
      grid=(NP // BSC,),
      in_specs=[
          pl.BlockSpec((W, BSC, D), lambda i: (0, i, 0)),
          pl.BlockSpec((W, BSC, D), lambda i: (0, i, 0)),
          full(W, D), full(W, D), full(W, D), full(W, D),
          full(2 * D, 4 * H), full(H, 4 * H), full(1, 4 * H),
          full(H, 4 * H), full(H, 4 * H), full(1, 4 * H),
          full(H, 128), full(1, 128),
      ],
      out_specs=pl.BlockSpec((BSC, 128), lambda i: (i, 0)),
      out_shape=jax.ShapeDtypeStruct((NP, 128), F32),
  )(h1, P2, g2, b2, m2, v2, W1, U1, b1, W2, U2, bb2, Wdp, bdp)


def _split_idx(base):
  two = base * 2
  stacked = jnp.stack([two, two + 1])
  return stacked.reshape(NC, W, NS, NSUP, CG, CH)


def kernel(X, edge_index, bn1_gamma, bn1_beta, bn1_mean, bn1_var,
           bn2_gamma, bn2_beta, bn2_mean, bn2_var,
           W1, U1, b1, W2, U2, b2, Wd, bd):
  src = edge_index[:, 0, :]
  dst = edge_index[:, 1, :]
  woff = jnp.arange(W, dtype=jnp.int32)[:, None]
  src1 = _split_idx(src + woff * N)
  src2 = _split_idx(src + woff * NP)
  dstr = dst.reshape(W, NS, NSUP, CG, CH)

  zrow = jnp.zeros((RPT, HD), F32)
  zdeg = jnp.zeros((RPT, DW), F32)
  half = jnp.full((CH, DW), 0.5, F32)

  P1, Dg = _make_mpnn(True)(X.reshape(W * N * NC, HD), src1, dstr,
                            zrow, zdeg, half)
  h1 = _bn1(P1, Dg, bn1_gamma, bn1_beta, bn1_mean, bn1_var)
  P2 = _make_mpnn(False)(h1.reshape(W * NP * NC, HD), src2, dstr, zrow)

  Wdp = jnp.pad(Wd, ((0, 0), (0, 127)))
  bdp = jnp.pad(bd, (0, 127))[None, :]
  out = _head(h1, P2, bn2_gamma, bn2_beta, bn2_mean, bn2_var,
              W1, U1, b1[None, :], W2, U2, b2[None, :], Wdp, bdp)
  return out[:N, :1]


# speedup vs baseline: 1.3773x; 1.0592x over previous
import functools

import jax
import jax.numpy as jnp
from jax import lax
from jax.experimental import pallas as pl
from jax.experimental.pallas import tpu as pltpu
from jax.experimental.pallas import tpu_sc as plsc

N = 10000
D = 128
E = 320000
W = 6
H = 128
EPS = 1e-3

NC = 2
NS = 16
HD = D // NC
NP = 10240
RPT = NP // NS
EPT = E // NS
CH = 125
CG = 80
NSUP = EPT // (CH * CG)
NBUF = 4
NIF = 3
DW = 16

BSB = 512
BSC = 512

F32 = jnp.float32


@functools.lru_cache(maxsize=None)
def _make_mpnn(with_deg):
  mesh = plsc.VectorSubcoreMesh(core_axis_name="c", subcore_axis_name="s")
  out_type = [jax.ShapeDtypeStruct((W, NP, D), F32)]
  scratch = [
      pltpu.VMEM((CG, CH), jnp.int32),
      pltpu.VMEM((CG, CH), jnp.int32),
      pltpu.VMEM((NBUF, CH, HD), F32),
      pltpu.VMEM_SHARED((NP, HD), F32),
      pltpu.SemaphoreType.DMA,
      pltpu.SemaphoreType.DMA,
      pltpu.SemaphoreType.DMA,
  ]
  if with_deg:
    out_type.append(jax.ShapeDtypeStruct((NC, W, NP, DW), F32))
    scratch += [pltpu.VMEM((CH, DW), F32), pltpu.VMEM_SHARED((NP, DW), F32)]

  def body(table, src_h, dst_h, zrow_h, *rest):
    if with_deg:
      (zdeg_h, half_h, out_p, out_d, src_v, dst_v, rows_v, acc,
       sem0, sem1, sem2, ones_v, dacc) = rest
    else:
      (out_p, src_v, dst_v, rows_v, acc, sem0, sem1, sem2) = rest
    sems = (sem0, sem1, sem2)
    c = lax.axis_index("c")
    s = lax.axis_index("s")
    row0 = s * RPT
    if with_deg:
      pltpu.sync_copy(half_h, ones_v)

    def win_body(w, carry):
      pltpu.sync_copy(zrow_h, acc.at[pl.ds(row0, RPT)])
      if with_deg:
        pltpu.sync_copy(zdeg_h, dacc.at[pl.ds(row0, RPT)])
      plsc.subcore_barrier()

      def chunk_body(j, carry2):
        pltpu.sync_copy(src_h.at[c, w, s, j], src_v)
        pltpu.sync_copy(dst_h.at[w, s, j], dst_v)
        descs = [None] * CG
        for jj in range(NIF):
          descs[jj] = pltpu.async_copy(
              table.at[src_v.at[jj]], rows_v.at[jj % NBUF], sems[jj % NIF])
        for jj in range(CG):
          descs[jj].wait()
          if jj + NIF < CG:
            descs[jj + NIF] = pltpu.async_copy(
                table.at[src_v.at[jj + NIF]], rows_v.at[(jj + NIF) % NBUF],
                sems[jj % NIF])
          pltpu.sync_copy(rows_v.at[jj % NBUF], acc.at[dst_v.at[jj]],
                          add=True)
          if with_deg:
            pltpu.sync_copy(ones_v, dacc.at[dst_v.at[jj]], add=True)
        return carry2

      lax.fori_loop(0, NSUP, chunk_body, 0)
      plsc.subcore_barrier()
      pltpu.sync_copy(acc.at[pl.ds(row0, RPT)],
                      out_p.at[w, pl.ds(row0, RPT), pl.ds(c * HD, HD)])
      if with_deg:
        pltpu.sync_copy(dacc.at[pl.ds(row0, RPT)],
                        out_d.at[c, w, pl.ds(row0, RPT)])
      return carry

    lax.fori_loop(0, W, win_body, 0)

  return pl.kernel(
      body,
      out_type=tuple(out_type) if with_deg else out_type[0],
      mesh=mesh,
      compiler_params=pltpu.CompilerParams(use_tc_tiling_on_sc=False),
      scratch_types=scratch,
  )


def _bn1_body(p_ref, d_ref, g_ref, b_ref, m_ref, v_ref, o_ref):
  p = p_ref[0]
  deg = d_ref[0, 0] + d_ref[1, 0]
  degc = jnp.maximum(deg[:, :1], 1.0)
  h = jnp.maximum(p / degc, 0.0)
  o_ref[0] = ((h - m_ref[0, 0]) * lax.rsqrt(v_ref[0, 0] + EPS) * g_ref[0, 0]
              + b_ref[0, 0])


def _bn1(P, Dg, g, b, m, v):
  g, b, m, v = (x[:, None, :] for x in (g, b, m, v))
  pspec = pl.BlockSpec((1, BSB, D), lambda w, i: (w, i, 0))
  dspec = pl.BlockSpec((NC, 1, BSB, DW), lambda w, i: (0, w, i, 0))
  wspec = pl.BlockSpec((1, 1, D), lambda w, i: (w, 0, 0))
  return pl.pallas_call(
      _bn1_body,
      grid=(W, NP // BSB),
      in_specs=[pspec, dspec, wspec, wspec, wspec, wspec],
      out_specs=pl.BlockSpec((1, BSB, D), lambda w, i: (w, i, 0)),
      out_shape=jax.ShapeDtypeStruct((W, NP, D), F32),
  )(P, Dg, g, b, m, v)


def _sigmoid(x):
  return 1.0 / (1.0 + jnp.exp(-x))


def _head_body(h1_ref, p2_ref, g2_ref, b2_ref, m2_ref, v2_ref,
               w1_ref, u1_ref, b1_ref, w2_ref, u2_ref, bb2_ref,
               wd_ref, bd_ref, o_ref):
  xs = []
  for w in range(W):
    h2 = jnp.maximum(p2_ref[w], 0.0)
    h2 = ((h2 - m2_ref[w]) * lax.rsqrt(v2_ref[w] + EPS) * g2_ref[w]
          + b2_ref[w])
    xs.append(jnp.concatenate([h1_ref[w], h2], axis=1))

  w1 = w1_ref[...]
  u1 = u1_ref[...]
  b1 = b1_ref[0]
  h = jnp.zeros((BSC, H), F32)
  c = jnp.zeros((BSC, H), F32)
  hs = []
  for t in range(W):
    z = (jnp.dot(xs[t], w1, preferred_element_type=F32)
         + jnp.dot(h, u1, preferred_element_type=F32) + b1)
    c = _sigmoid(z[:, H:2 * H]) * c + _sigmoid(z[:, :H]) * jnp.tanh(
        z[:, 2 * H:3 * H])
    h = _sigmoid(z[:, 3 * H:]) * jnp.tanh(c)
    hs.append(h)

  w2 = w2_ref[...]
  u2 = u2_ref[...]
  b2 = bb2_ref[0]
  h = jnp.zeros((BSC, H), F32)
  c = jnp.zeros((BSC, H), F32)
  for t in range(W):
    z = (jnp.dot(hs[t], w2, preferred_element_type=F32)
         + jnp.dot(h, u2, preferred_element_type=F32) + b2)
    c = _sigmoid(z[:, H:2 * H]) * c + _sigmoid(z[:, :H]) * jnp.tanh(
        z[:, 2 * H:3 * H])
    h = _sigmoid(z[:, 3 * H:]) * jnp.tanh(c)

  o_ref[...] = jnp.maximum(
      jnp.dot(h, wd_ref[...], preferred_element_type=F32) + bd_ref[0], 0.0)


def _head(h1, P2, g2, b2, m2, v2, W1, U1, b1, W2, U2, bb2, Wdp, bdp):
  full = lambda *shape: pl.BlockSpec(shape, lambda i: (0,) * len(shape))
  return pl.pallas_call(
      _head_body,
      grid=(NP // BSC,),
      in_specs=[
          pl.BlockSpec((W, BSC, D), lambda i: (0, i, 0)),
          pl.BlockSpec((W, BSC, D), lambda i: (0, i, 0)),
          full(W, D), full(W, D), full(W, D), full(W, D),
          full(2 * D, 4 * H), full(H, 4 * H), full(1, 4 * H),
          full(H, 4 * H), full(H, 4 * H), full(1, 4 * H),
          full(H, 128), full(1, 128),
      ],
      out_specs=pl.BlockSpec((BSC, 128), lambda i: (i, 0)),
      out_shape=jax.ShapeDtypeStruct((NP, 128), F32),
  )(h1, P2, g2, b2, m2, v2, W1, U1, b1, W2, U2, bb2, Wdp, bdp)


def _split_idx(base):
  two = base * 2
  stacked = jnp.stack([two, two + 1])
  return stacked.reshape(NC, W, NS, NSUP, CG, CH)


def kernel(X, edge_index, bn1_gamma, bn1_beta, bn1_mean, bn1_var,
           bn2_gamma, bn2_beta, bn2_mean, bn2_var,
           W1, U1, b1, W2, U2, b2, Wd, bd):
  src = edge_index[:, 0, :]
  dst = edge_index[:, 1, :]
  woff = jnp.arange(W, dtype=jnp.int32)[:, None]
  src1 = _split_idx(src + woff * N)
  src2 = _split_idx(src + woff * NP)
  dstr = dst.reshape(W, NS, NSUP, CG, CH)

  zrow = jnp.zeros((RPT, HD), F32)
  zdeg = jnp.zeros((RPT, DW), F32)
  half = jnp.full((CH, DW), 0.5, F32)

  P1, Dg = _make_mpnn(True)(X.reshape(W * N * NC, HD), src1, dstr,
                            zrow, zdeg, half)
  h1 = _bn1(P1, Dg, bn1_gamma, bn1_beta, bn1_mean, bn1_var)
  P2 = _make_mpnn(False)(h1.reshape(W * NP * NC, HD), src2, dstr, zrow)

  Wdp = jnp.pad(Wd, ((0, 0), (0, 127)))
  bdp = jnp.pad(bd, (0, 127))[None, :]
  out = _head(h1, P2, bn2_gamma, bn2_beta, bn2_mean, bn2_var,
              W1, U1, b1[None, :], W2, U2, b2[None, :], Wdp, bdp)
  return out[:N, :1]


# speedup vs baseline: 1.4006x; 1.0169x over previous
import functools

import jax
import jax.numpy as jnp
from jax import lax
from jax.experimental import pallas as pl
from jax.experimental.pallas import tpu as pltpu
from jax.experimental.pallas import tpu_sc as plsc

N = 10000
D = 128
E = 320000
W = 6
H = 128
EPS = 1e-3

NC = 2
NS = 16
HD = D // NC
NP = 10240
RPT = NP // NS
EPT = E // NS
CH = 125
CG = 160
NSUP = EPT // (CH * CG)
NBUF = 4
NIF = 3
DW = 16

BSB = 512
BSC = 512

F32 = jnp.float32


@functools.lru_cache(maxsize=None)
def _make_mpnn(with_deg):
  mesh = plsc.VectorSubcoreMesh(core_axis_name="c", subcore_axis_name="s")
  out_type = [jax.ShapeDtypeStruct((W, NP, D), F32)]
  scratch = [
      pltpu.VMEM((CG, CH), jnp.int32),
      pltpu.VMEM((CG, CH), jnp.int32),
      pltpu.VMEM((NBUF, CH, HD), F32),
      pltpu.VMEM_SHARED((NP, HD), F32),
      pltpu.SemaphoreType.DMA,
      pltpu.SemaphoreType.DMA,
      pltpu.SemaphoreType.DMA,
  ]
  if with_deg:
    out_type.append(jax.ShapeDtypeStruct((NC, W, NP, DW), F32))
    scratch += [pltpu.VMEM((CH, DW), F32), pltpu.VMEM_SHARED((NP, DW), F32)]

  def body(table, src_h, dst_h, zrow_h, *rest):
    if with_deg:
      (zdeg_h, half_h, out_p, out_d, src_v, dst_v, rows_v, acc,
       sem0, sem1, sem2, ones_v, dacc) = rest
    else:
      (out_p, src_v, dst_v, rows_v, acc, sem0, sem1, sem2) = rest
    sems = (sem0, sem1, sem2)
    c = lax.axis_index("c")
    s = lax.axis_index("s")
    row0 = s * RPT
    if with_deg:
      pltpu.sync_copy(half_h, ones_v)

    def win_body(w, carry):
      pltpu.sync_copy(zrow_h, acc.at[pl.ds(row0, RPT)])
      if with_deg:
        pltpu.sync_copy(zdeg_h, dacc.at[pl.ds(row0, RPT)])
      plsc.subcore_barrier()

      def chunk_body(j, carry2):
        pltpu.sync_copy(src_h.at[c, w, s, j], src_v)
        pltpu.sync_copy(dst_h.at[w, s, j], dst_v)
        descs = [None] * CG
        for jj in range(NIF):
          descs[jj] = pltpu.async_copy(
              table.at[src_v.at[jj]], rows_v.at[jj % NBUF], sems[jj % NIF])
        for jj in range(CG):
          descs[jj].wait()
          if jj + NIF < CG:
            descs[jj + NIF] = pltpu.async_copy(
                table.at[src_v.at[jj + NIF]], rows_v.at[(jj + NIF) % NBUF],
                sems[jj % NIF])
          pltpu.sync_copy(rows_v.at[jj % NBUF], acc.at[dst_v.at[jj]],
                          add=True)
          if with_deg:
            pltpu.sync_copy(ones_v, dacc.at[dst_v.at[jj]], add=True)
        return carry2

      lax.fori_loop(0, NSUP, chunk_body, 0)
      plsc.subcore_barrier()
      pltpu.sync_copy(acc.at[pl.ds(row0, RPT)],
                      out_p.at[w, pl.ds(row0, RPT), pl.ds(c * HD, HD)])
      if with_deg:
        pltpu.sync_copy(dacc.at[pl.ds(row0, RPT)],
                        out_d.at[c, w, pl.ds(row0, RPT)])
      return carry

    lax.fori_loop(0, W, win_body, 0)

  return pl.kernel(
      body,
      out_type=tuple(out_type) if with_deg else out_type[0],
      mesh=mesh,
      compiler_params=pltpu.CompilerParams(use_tc_tiling_on_sc=False),
      scratch_types=scratch,
  )


def _bn1_body(p_ref, d_ref, g_ref, b_ref, m_ref, v_ref, o_ref):
  p = p_ref[0]
  deg = d_ref[0, 0] + d_ref[1, 0]
  degc = jnp.maximum(deg[:, :1], 1.0)
  h = jnp.maximum(p / degc, 0.0)
  o_ref[0] = ((h - m_ref[0, 0]) * lax.rsqrt(v_ref[0, 0] + EPS) * g_ref[0, 0]
              + b_ref[0, 0])


def _bn1(P, Dg, g, b, m, v):
  g, b, m, v = (x[:, None, :] for x in (g, b, m, v))
  pspec = pl.BlockSpec((1, BSB, D), lambda w, i: (w, i, 0))
  dspec = pl.BlockSpec((NC, 1, BSB, DW), lambda w, i: (0, w, i, 0))
  wspec = pl.BlockSpec((1, 1, D), lambda w, i: (w, 0, 0))
  return pl.pallas_call(
      _bn1_body,
      grid=(W, NP // BSB),
      in_specs=[pspec, dspec, wspec, wspec, wspec, wspec],
      out_specs=pl.BlockSpec((1, BSB, D), lambda w, i: (w, i, 0)),
      out_shape=jax.ShapeDtypeStruct((W, NP, D), F32),
  )(P, Dg, g, b, m, v)


def _sigmoid(x):
  return 1.0 / (1.0 + jnp.exp(-x))


def _head_body(h1_ref, p2_ref, g2_ref, b2_ref, m2_ref, v2_ref,
               w1_ref, u1_ref, b1_ref, w2_ref, u2_ref, bb2_ref,
               wd_ref, bd_ref, o_ref):
  xs = []
  for w in range(W):
    h2 = jnp.maximum(p2_ref[w], 0.0)
    h2 = ((h2 - m2_ref[w]) * lax.rsqrt(v2_ref[w] + EPS) * g2_ref[w]
          + b2_ref[w])
    xs.append(jnp.concatenate([h1_ref[w], h2], axis=1))

  w1 = w1_ref[...]
  u1 = u1_ref[...]
  b1 = b1_ref[0]
  h = jnp.zeros((BSC, H), F32)
  c = jnp.zeros((BSC, H), F32)
  hs = []
  for t in range(W):
    z = (jnp.dot(xs[t], w1, preferred_element_type=F32)
         + jnp.dot(h, u1, preferred_element_type=F32) + b1)
    c = _sigmoid(z[:, H:2 * H]) * c + _sigmoid(z[:, :H]) * jnp.tanh(
        z[:, 2 * H:3 * H])
    h = _sigmoid(z[:, 3 * H:]) * jnp.tanh(c)
    hs.append(h)

  w2 = w2_ref[...]
  u2 = u2_ref[...]
  b2 = bb2_ref[0]
  h = jnp.zeros((BSC, H), F32)
  c = jnp.zeros((BSC, H), F32)
  for t in range(W):
    z = (jnp.dot(hs[t], w2, preferred_element_type=F32)
         + jnp.dot(h, u2, preferred_element_type=F32) + b2)
    c = _sigmoid(z[:, H:2 * H]) * c + _sigmoid(z[:, :H]) * jnp.tanh(
        z[:, 2 * H:3 * H])
    h = _sigmoid(z[:, 3 * H:]) * jnp.tanh(c)

  o_ref[...] = jnp.maximum(
      jnp.dot(h, wd_ref[...], preferred_element_type=F32) + bd_ref[0], 0.0)


def _head(h1, P2, g2, b2, m2, v2, W1, U1, b1, W2, U2, bb2, Wdp, bdp):
  full = lambda *shape: pl.BlockSpec(shape, lambda i: (0,) * len(shape))
  return pl.pallas_call(
      _head_body,
      grid=(NP // BSC,),
      in_specs=[
          pl.BlockSpec((W, BSC, D), lambda i: (0, i, 0)),
          pl.BlockSpec((W, BSC, D), lambda i: (0, i, 0)),
          full(W, D), full(W, D), full(W, D), full(W, D),
          full(2 * D, 4 * H), full(H, 4 * H), full(1, 4 * H),
          full(H, 4 * H), full(H, 4 * H), full(1, 4 * H),
          full(H, 128), full(1, 128),
      ],
      out_specs=pl.BlockSpec((BSC, 128), lambda i: (i, 0)),
      out_shape=jax.ShapeDtypeStruct((NP, 128), F32),
  )(h1, P2, g2, b2, m2, v2, W1, U1, b1, W2, U2, bb2, Wdp, bdp)


def _split_idx(base):
  two = base * 2
  stacked = jnp.stack([two, two + 1])
  return stacked.reshape(NC, W, NS, NSUP, CG, CH)


def kernel(X, edge_index, bn1_gamma, bn1_beta, bn1_mean, bn1_var,
           bn2_gamma, bn2_beta, bn2_mean, bn2_var,
           W1, U1, b1, W2, U2, b2, Wd, bd):
  src = edge_index[:, 0, :]
  dst = edge_index[:, 1, :]
  woff = jnp.arange(W, dtype=jnp.int32)[:, None]
  src1 = _split_idx(src + woff * N)
  src2 = _split_idx(src + woff * NP)
  dstr = dst.reshape(W, NS, NSUP, CG, CH)

  zrow = jnp.zeros((RPT, HD), F32)
  zdeg = jnp.zeros((RPT, DW), F32)
  half = jnp.full((CH, DW), 0.5, F32)

  P1, Dg = _make_mpnn(True)(X.reshape(W * N * NC, HD), src1, dstr,
                            zrow, zdeg, half)
  h1 = _bn1(P1, Dg, bn1_gamma, bn1_beta, bn1_mean, bn1_var)
  P2 = _make_mpnn(False)(h1.reshape(W * NP * NC, HD), src2, dstr, zrow)

  Wdp = jnp.pad(Wd, ((0, 0), (0, 127)))
  bdp = jnp.pad(bd, (0, 127))[None, :]
  out = _head(h1, P2, bn2_gamma, bn2_beta, bn2_mean, bn2_var,
              W1, U1, b1[None, :], W2, U2, b2[None, :], Wdp, bdp)
  return out[:N, :1]
